# baseline (speedup 1.0000x reference)
import jax
import jax.numpy as jnp
from jax.experimental import pallas as pl


_GRID = 1


def _copy_body(se_in, rw_in, se_out, rw_out):
    se_out[...] = se_in[...]
    rw_out[...] = rw_in[...]


def kernel(hidden_states, selected_experts, routing_weights):
    rows = hidden_states.shape[0] * hidden_states.shape[1]
    k = selected_experts.shape[1]
    out_dtype = hidden_states.dtype

    se_t = selected_experts.T
    rw_t = routing_weights.astype(out_dtype).T

    block = rows // _GRID
    spec = pl.BlockSpec((k, block), lambda i: (0, i))

    se_o, rw_o = pl.pallas_call(
        _copy_body,
        grid=(_GRID,),
        in_specs=[spec, spec],
        out_specs=[spec, spec],
        out_shape=[
            jax.ShapeDtypeStruct((k, rows), selected_experts.dtype),
            jax.ShapeDtypeStruct((k, rows), out_dtype),
        ],
    )(se_t, rw_t)

    return se_o.T, rw_o.T


# speedup vs baseline: 1.0791x; 1.0791x over previous
import jax
import jax.numpy as jnp
from jax.experimental import pallas as pl


_GRID = 2


def _copy_body(se_in, rw_in, se_out, rw_out):
    se_out[...] = se_in[...]
    rw_out[...] = rw_in[...]


def kernel(hidden_states, selected_experts, routing_weights):
    rows = hidden_states.shape[0] * hidden_states.shape[1]
    k = selected_experts.shape[1]
    out_dtype = hidden_states.dtype

    se_t = selected_experts.T
    rw_t = routing_weights.astype(out_dtype).T

    block = rows // _GRID
    spec = pl.BlockSpec((k, block), lambda i: (0, i))

    se_o, rw_o = pl.pallas_call(
        _copy_body,
        grid=(_GRID,),
        in_specs=[spec, spec],
        out_specs=[spec, spec],
        out_shape=[
            jax.ShapeDtypeStruct((k, rows), selected_experts.dtype),
            jax.ShapeDtypeStruct((k, rows), out_dtype),
        ],
    )(se_t, rw_t)

    return se_o.T, rw_o.T
